# baseline (speedup 1.0000x reference)
import functools

import jax
import jax.numpy as jnp
from jax import lax
from jax.experimental import pallas as pl
from jax.experimental.pallas import tpu as pltpu
from jax.experimental.pallas import tpu_sc as plsc

D_MODEL = 1024
D_INNER = 4096
N_EXPERTS = 8
TOP_K = 2

N_TOK = 8192
_GM = 256
_BM = 256
_NB = (N_TOK * TOP_K + N_EXPERTS * (_BM - 1) + _BM - 1) // _BM
_NSLOT = _NB * _BM
_NW = 32


def _cumsum0(a):
    n = a.shape[0]
    k = 1
    while k < n:
        a = a + jnp.pad(a, ((k, 0), (0, 0)))[:n]
        k *= 2
    return a


def _gating_body(x_ref, wg_ref, bg_ref,
                 top2_ref, var_ref, xbf_ref, pos_ref, cnt_ref,
                 sum_ref, sq_ref, run_ref):
    i = pl.program_id(0)
    nt = pl.num_programs(0)
    x = x_ref[...]
    xbf_ref[...] = x.astype(jnp.bfloat16)
    logits = jnp.dot(x, wg_ref[...], preferred_element_type=jnp.float32)
    logits = logits + bg_ref[...]
    m = jnp.max(logits, axis=1, keepdims=True)
    e = jnp.exp(logits - m)
    p = e / jnp.sum(e, axis=1, keepdims=True)
    col = jax.lax.broadcasted_iota(jnp.int32, p.shape, 1)
    p1 = jnp.max(p, axis=1, keepdims=True)
    i1 = jnp.min(jnp.where(p == p1, col, N_EXPERTS), axis=1, keepdims=True)
    pm = jnp.where(col == i1, -jnp.inf, p)
    p2 = jnp.max(pm, axis=1, keepdims=True)
    i2 = jnp.min(jnp.where(pm == p2, col, N_EXPERTS), axis=1, keepdims=True)
    top2_ref[...] = jnp.concatenate([i1, i2], axis=1)

    @pl.when(i == 0)
    def _():
        sum_ref[...] = jnp.zeros_like(sum_ref)
        sq_ref[...] = jnp.zeros_like(sq_ref)
        run_ref[...] = jnp.zeros_like(run_ref)

    sum_ref[...] += jnp.sum(p, axis=0, keepdims=True)
    sq_ref[...] += jnp.sum(p * p, axis=0, keepdims=True)

    oh = (col == i1).astype(jnp.int32) + (col == i2).astype(jnp.int32)
    cum = _cumsum0(oh)
    base = run_ref[...] + (cum - oh)
    pos0 = jnp.sum(jnp.where(col == i1, base, 0), axis=1, keepdims=True)
    pos1 = jnp.sum(jnp.where(col == i2, base, 0), axis=1, keepdims=True)
    pos_ref[...] = jnp.concatenate([pos0, pos1], axis=1)
    run_ref[...] += jnp.sum(oh, axis=0, keepdims=True)

    @pl.when(i == nt - 1)
    def _():
        n = nt * _GM
        mean = sum_ref[...] / n
        var_ref[...] = (sq_ref[...] - n * mean * mean) / (n - 1)
        cnt_ref[...] = run_ref[...]


def _gating(flat, Wg, bg):
    grid = (N_TOK // _GM,)
    return pl.pallas_call(
        _gating_body,
        grid=grid,
        in_specs=[
            pl.BlockSpec((_GM, D_MODEL), lambda i: (i, 0)),
            pl.BlockSpec((D_MODEL, N_EXPERTS), lambda i: (0, 0)),
            pl.BlockSpec((1, N_EXPERTS), lambda i: (0, 0)),
        ],
        out_specs=[
            pl.BlockSpec((_GM, TOP_K), lambda i: (i, 0)),
            pl.BlockSpec((1, N_EXPERTS), lambda i: (0, 0)),
            pl.BlockSpec((_GM, D_MODEL), lambda i: (i, 0)),
            pl.BlockSpec((_GM, TOP_K), lambda i: (i, 0)),
            pl.BlockSpec((1, N_EXPERTS), lambda i: (0, 0)),
        ],
        out_shape=[
            jax.ShapeDtypeStruct((N_TOK, TOP_K), jnp.int32),
            jax.ShapeDtypeStruct((1, N_EXPERTS), jnp.float32),
            jax.ShapeDtypeStruct((N_TOK, D_MODEL), jnp.bfloat16),
            jax.ShapeDtypeStruct((N_TOK, TOP_K), jnp.int32),
            jax.ShapeDtypeStruct((1, N_EXPERTS), jnp.int32),
        ],
        scratch_shapes=[
            pltpu.VMEM((1, N_EXPERTS), jnp.float32),
            pltpu.VMEM((1, N_EXPERTS), jnp.float32),
            pltpu.VMEM((1, N_EXPERTS), jnp.int32),
        ],
    )(flat, Wg, bg.reshape(1, N_EXPERTS))


_CH2 = 64
_CH4 = 16


@functools.cache
def _make_dispatch_scatter():
    mesh = plsc.VectorSubcoreMesh(core_axis_name="c", subcore_axis_name="s")

    @functools.partial(
        pl.kernel,
        mesh=mesh,
        out_type=jax.ShapeDtypeStruct((_NSLOT, 4, 128), jnp.int32),
        scratch_types=[
            pltpu.VMEM((_CH2,), jnp.int32),
            pltpu.VMEM((_CH2, 4, 128), jnp.int32),
            pltpu.SemaphoreType.DMA,
        ],
    )
    def k(flat_hbm, dest_hbm, xs_hbm, didx_v, rows_v, sem):
        c = lax.axis_index("c")
        s = lax.axis_index("s")
        wid = s * 2 + c
        half = wid // 16
        seg = wid % 16
        base = seg * (N_TOK // 16)
        for j in range(N_TOK // 16 // _CH2):
            off = base + j * _CH2
            pltpu.sync_copy(flat_hbm.at[pl.ds(off, _CH2)], rows_v)
            pltpu.sync_copy(dest_hbm.at[pl.ds(half * N_TOK + off, _CH2)],
                            didx_v)
            pltpu.async_copy(rows_v, xs_hbm.at[didx_v], sem).wait()

    return k


def _gmm_body(be_map_ref, xs_ref, w_ref, b_ref, ys_ref):
    acc = jnp.dot(xs_ref[...], w_ref[0],
                  preferred_element_type=jnp.float32)
    ys_ref[...] = (acc + b_ref[0]).astype(jnp.bfloat16)


def _gmm(block_expert, xs, We_bf, be):
    grid_spec = pltpu.PrefetchScalarGridSpec(
        num_scalar_prefetch=1,
        grid=(_NB,),
        in_specs=[
            pl.BlockSpec((_BM, D_MODEL), lambda b, bm: (b, 0)),
            pl.BlockSpec((1, D_MODEL, D_INNER), lambda b, bm: (bm[b], 0, 0)),
            pl.BlockSpec((1, 1, D_INNER), lambda b, bm: (bm[b], 0, 0)),
        ],
        out_specs=pl.BlockSpec((_BM, D_INNER), lambda b, bm: (b, 0)),
    )
    return pl.pallas_call(
        _gmm_body,
        grid_spec=grid_spec,
        out_shape=jax.ShapeDtypeStruct((_NSLOT, D_INNER), jnp.bfloat16),
        compiler_params=pltpu.CompilerParams(
            dimension_semantics=("arbitrary",),
        ),
    )(block_expert, xs, We_bf, be.reshape(N_EXPERTS, 1, D_INNER))


@functools.cache
def _make_pair_gather():
    mesh = plsc.VectorSubcoreMesh(core_axis_name="c", subcore_axis_name="s")

    @functools.partial(
        pl.kernel,
        mesh=mesh,
        out_type=[
            jax.ShapeDtypeStruct((N_TOK, 16, 128), jnp.int32),
            jax.ShapeDtypeStruct((N_TOK, 16, 128), jnp.int32),
        ],
        scratch_types=[
            pltpu.VMEM((_CH4,), jnp.int32),
            pltpu.VMEM((_CH4,), jnp.int32),
            pltpu.VMEM((_CH4, 16, 128), jnp.int32),
            pltpu.VMEM((_CH4, 16, 128), jnp.int32),
            pltpu.SemaphoreType.DMA,
        ],
    )
    def k(ys_hbm, dest_hbm, y0_hbm, y1_hbm,
          idx0_v, idx1_v, b0_v, b1_v, sem):
        c = lax.axis_index("c")
        s = lax.axis_index("s")
        wid = s * 2 + c
        base = wid * (N_TOK // _NW)
        for j in range(N_TOK // _NW // _CH4):
            off = base + j * _CH4
            pltpu.sync_copy(dest_hbm.at[pl.ds(off, _CH4)], idx0_v)
            pltpu.sync_copy(dest_hbm.at[pl.ds(N_TOK + off, _CH4)], idx1_v)
            cp0 = pltpu.async_copy(ys_hbm.at[idx0_v], b0_v, sem)
            cp1 = pltpu.async_copy(ys_hbm.at[idx1_v], b1_v, sem)
            cp0.wait()
            cp1.wait()
            pltpu.sync_copy(b0_v, y0_hbm.at[pl.ds(off, _CH4)])
            pltpu.sync_copy(b1_v, y1_hbm.at[pl.ds(off, _CH4)])

    return k


def _combine_body(y0_ref, y1_ref, out_ref):
    out_ref[...] = (y0_ref[...].astype(jnp.float32)
                    + y1_ref[...].astype(jnp.float32)) * (1.0 / TOP_K)


def _combine(y0, y1):
    bm = 512
    return pl.pallas_call(
        _combine_body,
        grid=(N_TOK // bm,),
        in_specs=[
            pl.BlockSpec((bm, D_INNER), lambda i: (i, 0)),
            pl.BlockSpec((bm, D_INNER), lambda i: (i, 0)),
        ],
        out_specs=pl.BlockSpec((bm, D_INNER), lambda i: (i, 0)),
        out_shape=jax.ShapeDtypeStruct((N_TOK, D_INNER), jnp.float32),
    )(y0, y1)


def kernel(sequences, We, be, Wg, bg):
    N, P, D = sequences.shape
    flat = sequences.reshape(-1, D)
    top2, var, flat_bf, pos, counts = _gating(flat, Wg, bg)

    counts = counts.reshape(N_EXPERTS)
    padded = ((counts + _BM - 1) // _BM) * _BM
    cum = jnp.cumsum(padded)
    pstart = cum - padded
    dest = pstart[top2] + pos
    dest_flat = dest.T.reshape(-1)
    block_expert = jnp.clip(
        jnp.searchsorted(cum, jnp.arange(_NB, dtype=jnp.int32) * _BM,
                         side="right"),
        0, N_EXPERTS - 1).astype(jnp.int32)

    We_bf = We.astype(jnp.bfloat16)
    flat_i = lax.bitcast_convert_type(
        flat_bf.reshape(N_TOK, D_MODEL // 2, 2), jnp.int32
    ).reshape(N_TOK, 4, 128)
    xs_i = _make_dispatch_scatter()(flat_i, dest_flat)
    xs_bf = lax.bitcast_convert_type(
        xs_i.reshape(_NSLOT, D_MODEL // 2, 1), jnp.bfloat16
    ).reshape(_NSLOT, D_MODEL)
    ys = _gmm(block_expert, xs_bf, We_bf, be)
    ys_i = lax.bitcast_convert_type(
        ys.reshape(_NSLOT, D_INNER // 2, 2), jnp.int32
    ).reshape(_NSLOT, 16, 128)
    y0, y1 = _make_pair_gather()(ys_i, dest_flat)
    y0_bf = lax.bitcast_convert_type(
        y0.reshape(N_TOK, D_INNER // 2, 1), jnp.bfloat16
    ).reshape(N_TOK, D_INNER)
    y1_bf = lax.bitcast_convert_type(
        y1.reshape(N_TOK, D_INNER // 2, 1), jnp.bfloat16
    ).reshape(N_TOK, D_INNER)
    out = _combine(y0_bf, y1_bf)
    return (out.reshape(N, P, -1), var.reshape(N_EXPERTS))


# speedup vs baseline: 6.1601x; 6.1601x over previous
import functools

import jax
import jax.numpy as jnp
from jax import lax
from jax.experimental import pallas as pl
from jax.experimental.pallas import tpu as pltpu
from jax.experimental.pallas import tpu_sc as plsc

D_MODEL = 1024
D_INNER = 4096
N_EXPERTS = 8
TOP_K = 2

N_TOK = 8192
_GM = 256
_BM = 256
_NB = (N_TOK * TOP_K + N_EXPERTS * (_BM - 1) + _BM - 1) // _BM
_NSLOT = _NB * _BM
_NW = 32


def _cumsum0(a):
    n = a.shape[0]
    k = 1
    while k < n:
        a = a + jnp.pad(a, ((k, 0), (0, 0)))[:n]
        k *= 2
    return a


def _pack_pair(a, b):
    au = lax.bitcast_convert_type(a, jnp.uint32)
    bu = lax.bitcast_convert_type(b, jnp.uint32)
    ar = (au + 0x7FFF + ((au >> 16) & 1)) >> 16
    br = (bu + 0x7FFF + ((bu >> 16) & 1)) & jnp.uint32(0xFFFF0000)
    return lax.bitcast_convert_type(ar | br, jnp.int32)


def _unpack_pair(w):
    wu = lax.bitcast_convert_type(w, jnp.uint32)
    lo = lax.bitcast_convert_type(wu << 16, jnp.float32)
    hi = lax.bitcast_convert_type(wu & jnp.uint32(0xFFFF0000), jnp.float32)
    return lo, hi


def _gating_body(x_ref, wg_ref, bg_ref,
                 top2_ref, var_ref, xp_ref, pos_ref, cnt_ref,
                 sum_ref, sq_ref, run_ref):
    i = pl.program_id(0)
    nt = pl.num_programs(0)
    x = x_ref[...]
    xp_ref[...] = _pack_pair(x[:, :D_MODEL // 2], x[:, D_MODEL // 2:])
    logits = jnp.dot(x, wg_ref[...], preferred_element_type=jnp.float32)
    logits = logits + bg_ref[...]
    m = jnp.max(logits, axis=1, keepdims=True)
    e = jnp.exp(logits - m)
    p = e / jnp.sum(e, axis=1, keepdims=True)
    col = jax.lax.broadcasted_iota(jnp.int32, p.shape, 1)
    p1 = jnp.max(p, axis=1, keepdims=True)
    i1 = jnp.min(jnp.where(p == p1, col, N_EXPERTS), axis=1, keepdims=True)
    pm = jnp.where(col == i1, -jnp.inf, p)
    p2 = jnp.max(pm, axis=1, keepdims=True)
    i2 = jnp.min(jnp.where(pm == p2, col, N_EXPERTS), axis=1, keepdims=True)
    top2_ref[...] = jnp.concatenate([i1, i2], axis=1)

    @pl.when(i == 0)
    def _():
        sum_ref[...] = jnp.zeros_like(sum_ref)
        sq_ref[...] = jnp.zeros_like(sq_ref)
        run_ref[...] = jnp.zeros_like(run_ref)

    sum_ref[...] += jnp.sum(p, axis=0, keepdims=True)
    sq_ref[...] += jnp.sum(p * p, axis=0, keepdims=True)

    oh = (col == i1).astype(jnp.int32) + (col == i2).astype(jnp.int32)
    cum = _cumsum0(oh)
    base = run_ref[...] + (cum - oh)
    pos0 = jnp.sum(jnp.where(col == i1, base, 0), axis=1, keepdims=True)
    pos1 = jnp.sum(jnp.where(col == i2, base, 0), axis=1, keepdims=True)
    pos_ref[...] = jnp.concatenate([pos0, pos1], axis=1)
    run_ref[...] += jnp.sum(oh, axis=0, keepdims=True)

    @pl.when(i == nt - 1)
    def _():
        n = nt * _GM
        mean = sum_ref[...] / n
        var_ref[...] = (sq_ref[...] - n * mean * mean) / (n - 1)
        cnt_ref[...] = run_ref[...]


def _gating(flat, Wg, bg):
    grid = (N_TOK // _GM,)
    return pl.pallas_call(
        _gating_body,
        grid=grid,
        in_specs=[
            pl.BlockSpec((_GM, D_MODEL), lambda i: (i, 0)),
            pl.BlockSpec((D_MODEL, N_EXPERTS), lambda i: (0, 0)),
            pl.BlockSpec((1, N_EXPERTS), lambda i: (0, 0)),
        ],
        out_specs=[
            pl.BlockSpec((_GM, TOP_K), lambda i: (i, 0)),
            pl.BlockSpec((1, N_EXPERTS), lambda i: (0, 0)),
            pl.BlockSpec((_GM, D_MODEL // 2), lambda i: (i, 0)),
            pl.BlockSpec((_GM, TOP_K), lambda i: (i, 0)),
            pl.BlockSpec((1, N_EXPERTS), lambda i: (0, 0)),
        ],
        out_shape=[
            jax.ShapeDtypeStruct((N_TOK, TOP_K), jnp.int32),
            jax.ShapeDtypeStruct((1, N_EXPERTS), jnp.float32),
            jax.ShapeDtypeStruct((N_TOK, D_MODEL // 2), jnp.int32),
            jax.ShapeDtypeStruct((N_TOK, TOP_K), jnp.int32),
            jax.ShapeDtypeStruct((1, N_EXPERTS), jnp.int32),
        ],
        scratch_shapes=[
            pltpu.VMEM((1, N_EXPERTS), jnp.float32),
            pltpu.VMEM((1, N_EXPERTS), jnp.float32),
            pltpu.VMEM((1, N_EXPERTS), jnp.int32),
        ],
    )(flat, Wg, bg.reshape(1, N_EXPERTS))


_CH2 = 64
_CH4 = 16


@functools.cache
def _make_dispatch_scatter():
    mesh = plsc.VectorSubcoreMesh(core_axis_name="c", subcore_axis_name="s")

    @functools.partial(
        pl.kernel,
        mesh=mesh,
        out_type=jax.ShapeDtypeStruct((_NSLOT, D_MODEL // 2), jnp.int32),
        scratch_types=[
            pltpu.VMEM((_CH2,), jnp.int32),
            pltpu.VMEM((_CH2, D_MODEL // 2), jnp.int32),
            pltpu.SemaphoreType.DMA,
        ],
    )
    def k(flat_hbm, dest_hbm, xs_hbm, didx_v, rows_v, sem):
        c = lax.axis_index("c")
        s = lax.axis_index("s")
        wid = s * 2 + c
        half = wid // 16
        seg = wid % 16
        base = seg * (N_TOK // 16)
        for j in range(N_TOK // 16 // _CH2):
            off = base + j * _CH2
            pltpu.sync_copy(flat_hbm.at[pl.ds(off, _CH2)], rows_v)
            pltpu.sync_copy(dest_hbm.at[pl.ds(half * N_TOK + off, _CH2)],
                            didx_v)
            pltpu.async_copy(rows_v, xs_hbm.at[didx_v], sem).wait()

    return k


def _gmm_body(be_map_ref, xs_ref, w_ref, b_ref, ys_ref):
    xlo, xhi = _unpack_pair(xs_ref[...])
    xbf = jnp.concatenate([xlo, xhi], axis=1).astype(jnp.bfloat16)
    acc = jnp.dot(xbf, w_ref[0], preferred_element_type=jnp.float32)
    acc = acc + b_ref[0]
    ys_ref[...] = _pack_pair(acc[:, :D_INNER // 2], acc[:, D_INNER // 2:])


def _gmm(block_expert, xs, We_bf, be):
    grid_spec = pltpu.PrefetchScalarGridSpec(
        num_scalar_prefetch=1,
        grid=(_NB,),
        in_specs=[
            pl.BlockSpec((_BM, D_MODEL // 2), lambda b, bm: (b, 0)),
            pl.BlockSpec((1, D_MODEL, D_INNER), lambda b, bm: (bm[b], 0, 0)),
            pl.BlockSpec((1, 1, D_INNER), lambda b, bm: (bm[b], 0, 0)),
        ],
        out_specs=pl.BlockSpec((_BM, D_INNER // 2), lambda b, bm: (b, 0)),
    )
    return pl.pallas_call(
        _gmm_body,
        grid_spec=grid_spec,
        out_shape=jax.ShapeDtypeStruct((_NSLOT, D_INNER // 2), jnp.int32),
        compiler_params=pltpu.CompilerParams(
            dimension_semantics=("arbitrary",),
        ),
    )(block_expert, xs, We_bf, be.reshape(N_EXPERTS, 1, D_INNER))


@functools.cache
def _make_pair_gather():
    mesh = plsc.VectorSubcoreMesh(core_axis_name="c", subcore_axis_name="s")

    @functools.partial(
        pl.kernel,
        mesh=mesh,
        out_type=[
            jax.ShapeDtypeStruct((N_TOK, D_INNER // 2), jnp.int32),
            jax.ShapeDtypeStruct((N_TOK, D_INNER // 2), jnp.int32),
        ],
        scratch_types=[
            pltpu.VMEM((_CH4,), jnp.int32),
            pltpu.VMEM((_CH4,), jnp.int32),
            pltpu.VMEM((_CH4, D_INNER // 2), jnp.int32),
            pltpu.VMEM((_CH4, D_INNER // 2), jnp.int32),
            pltpu.SemaphoreType.DMA,
        ],
    )
    def k(ys_hbm, dest_hbm, y0_hbm, y1_hbm,
          idx0_v, idx1_v, b0_v, b1_v, sem):
        c = lax.axis_index("c")
        s = lax.axis_index("s")
        wid = s * 2 + c
        base = wid * (N_TOK // _NW)
        for j in range(N_TOK // _NW // _CH4):
            off = base + j * _CH4
            pltpu.sync_copy(dest_hbm.at[pl.ds(off, _CH4)], idx0_v)
            pltpu.sync_copy(dest_hbm.at[pl.ds(N_TOK + off, _CH4)], idx1_v)
            cp0 = pltpu.async_copy(ys_hbm.at[idx0_v], b0_v, sem)
            cp1 = pltpu.async_copy(ys_hbm.at[idx1_v], b1_v, sem)
            cp0.wait()
            cp1.wait()
            pltpu.sync_copy(b0_v, y0_hbm.at[pl.ds(off, _CH4)])
            pltpu.sync_copy(b1_v, y1_hbm.at[pl.ds(off, _CH4)])

    return k


def _combine_body(y0_ref, y1_ref, out_ref):
    lo0, hi0 = _unpack_pair(y0_ref[...])
    lo1, hi1 = _unpack_pair(y1_ref[...])
    out_ref[...] = jnp.concatenate(
        [lo0 + lo1, hi0 + hi1], axis=1) * (1.0 / TOP_K)


def _combine(y0, y1):
    bm = 512
    return pl.pallas_call(
        _combine_body,
        grid=(N_TOK // bm,),
        in_specs=[
            pl.BlockSpec((bm, D_INNER // 2), lambda i: (i, 0)),
            pl.BlockSpec((bm, D_INNER // 2), lambda i: (i, 0)),
        ],
        out_specs=pl.BlockSpec((bm, D_INNER), lambda i: (i, 0)),
        out_shape=jax.ShapeDtypeStruct((N_TOK, D_INNER), jnp.float32),
    )(y0, y1)


def kernel(sequences, We, be, Wg, bg):
    N, P, D = sequences.shape
    flat = sequences.reshape(-1, D)
    top2, var, flat_p, pos, counts = _gating(flat, Wg, bg)

    counts = counts.reshape(N_EXPERTS)
    padded = ((counts + _BM - 1) // _BM) * _BM
    cum = jnp.cumsum(padded)
    pstart = cum - padded
    dest = pstart[top2] + pos
    dest_flat = dest.T.reshape(-1)
    block_expert = jnp.clip(
        jnp.searchsorted(cum, jnp.arange(_NB, dtype=jnp.int32) * _BM,
                         side="right"),
        0, N_EXPERTS - 1).astype(jnp.int32)

    We_bf = We.astype(jnp.bfloat16)
    xs = _make_dispatch_scatter()(flat_p, dest_flat)
    ys = _gmm(block_expert, xs, We_bf, be)
    y0, y1 = _make_pair_gather()(ys, dest_flat)
    out = _combine(y0, y1)
    return (out.reshape(N, P, -1), var.reshape(N_EXPERTS))


# speedup vs baseline: 6.5084x; 1.0565x over previous
import functools

import jax
import jax.numpy as jnp
from jax import lax
from jax.experimental import pallas as pl
from jax.experimental.pallas import tpu as pltpu
from jax.experimental.pallas import tpu_sc as plsc

D_MODEL = 1024
D_INNER = 4096
N_EXPERTS = 8
TOP_K = 2

N_TOK = 8192
_GM = 512
_BM = 256
_NB = (N_TOK * TOP_K + N_EXPERTS * (_BM - 1) + _BM - 1) // _BM
_NSLOT = _NB * _BM
_NW = 32


def _cumsum0(a):
    n = a.shape[0]
    k = 1
    while k < n:
        a = a + jnp.pad(a, ((k, 0), (0, 0)))[:n]
        k *= 2
    return a


def _pack_pair(a, b):
    au = lax.bitcast_convert_type(a, jnp.uint32)
    bu = lax.bitcast_convert_type(b, jnp.uint32)
    ar = (au + 0x7FFF + ((au >> 16) & 1)) >> 16
    br = (bu + 0x7FFF + ((bu >> 16) & 1)) & jnp.uint32(0xFFFF0000)
    return lax.bitcast_convert_type(ar | br, jnp.int32)


def _unpack_pair(w):
    wu = lax.bitcast_convert_type(w, jnp.uint32)
    lo = lax.bitcast_convert_type(wu << 16, jnp.float32)
    hi = lax.bitcast_convert_type(wu & jnp.uint32(0xFFFF0000), jnp.float32)
    return lo, hi


def _gating_body(x_ref, wg_ref, bg_ref,
                 top2_ref, var_ref, xp_ref, pos_ref, cnt_ref,
                 sum_ref, sq_ref, run_ref):
    i = pl.program_id(0)
    nt = pl.num_programs(0)
    x = x_ref[...]
    xp_ref[...] = _pack_pair(x[:, :D_MODEL // 2], x[:, D_MODEL // 2:])
    logits = jnp.dot(x, wg_ref[...], preferred_element_type=jnp.float32)
    logits = logits + bg_ref[...]
    m = jnp.max(logits, axis=1, keepdims=True)
    e = jnp.exp(logits - m)
    p = e / jnp.sum(e, axis=1, keepdims=True)
    col = jax.lax.broadcasted_iota(jnp.int32, p.shape, 1)
    p1 = jnp.max(p, axis=1, keepdims=True)
    i1 = jnp.min(jnp.where(p == p1, col, N_EXPERTS), axis=1, keepdims=True)
    pm = jnp.where(col == i1, -jnp.inf, p)
    p2 = jnp.max(pm, axis=1, keepdims=True)
    i2 = jnp.min(jnp.where(pm == p2, col, N_EXPERTS), axis=1, keepdims=True)
    top2_ref[...] = jnp.concatenate([i1, i2], axis=1)

    @pl.when(i == 0)
    def _():
        sum_ref[...] = jnp.zeros_like(sum_ref)
        sq_ref[...] = jnp.zeros_like(sq_ref)
        run_ref[...] = jnp.zeros_like(run_ref)

    sum_ref[...] += jnp.sum(p, axis=0, keepdims=True)
    sq_ref[...] += jnp.sum(p * p, axis=0, keepdims=True)

    oh = (col == i1).astype(jnp.int32) + (col == i2).astype(jnp.int32)
    cum = _cumsum0(oh)
    base = run_ref[...] + (cum - oh)
    pos0 = jnp.sum(jnp.where(col == i1, base, 0), axis=1, keepdims=True)
    pos1 = jnp.sum(jnp.where(col == i2, base, 0), axis=1, keepdims=True)
    pos_ref[...] = jnp.concatenate([pos0, pos1], axis=1)
    run_ref[...] += jnp.sum(oh, axis=0, keepdims=True)

    @pl.when(i == nt - 1)
    def _():
        n = nt * _GM
        mean = sum_ref[...] / n
        var_ref[...] = (sq_ref[...] - n * mean * mean) / (n - 1)
        cnt_ref[...] = run_ref[...]


def _gating(flat, Wg, bg):
    grid = (N_TOK // _GM,)
    return pl.pallas_call(
        _gating_body,
        grid=grid,
        in_specs=[
            pl.BlockSpec((_GM, D_MODEL), lambda i: (i, 0)),
            pl.BlockSpec((D_MODEL, N_EXPERTS), lambda i: (0, 0)),
            pl.BlockSpec((1, N_EXPERTS), lambda i: (0, 0)),
        ],
        out_specs=[
            pl.BlockSpec((_GM, TOP_K), lambda i: (i, 0)),
            pl.BlockSpec((1, N_EXPERTS), lambda i: (0, 0)),
            pl.BlockSpec((_GM, D_MODEL // 2), lambda i: (i, 0)),
            pl.BlockSpec((_GM, TOP_K), lambda i: (i, 0)),
            pl.BlockSpec((1, N_EXPERTS), lambda i: (0, 0)),
        ],
        out_shape=[
            jax.ShapeDtypeStruct((N_TOK, TOP_K), jnp.int32),
            jax.ShapeDtypeStruct((1, N_EXPERTS), jnp.float32),
            jax.ShapeDtypeStruct((N_TOK, D_MODEL // 2), jnp.int32),
            jax.ShapeDtypeStruct((N_TOK, TOP_K), jnp.int32),
            jax.ShapeDtypeStruct((1, N_EXPERTS), jnp.int32),
        ],
        scratch_shapes=[
            pltpu.VMEM((1, N_EXPERTS), jnp.float32),
            pltpu.VMEM((1, N_EXPERTS), jnp.float32),
            pltpu.VMEM((1, N_EXPERTS), jnp.int32),
        ],
    )(flat, Wg, bg.reshape(1, N_EXPERTS))


_CH2 = 64
_CH4 = 8


@functools.cache
def _make_dispatch_scatter():
    mesh = plsc.VectorSubcoreMesh(core_axis_name="c", subcore_axis_name="s")

    nchunk = N_TOK // 16 // _CH2

    @functools.partial(
        pl.kernel,
        mesh=mesh,
        out_type=jax.ShapeDtypeStruct((_NSLOT, D_MODEL // 2), jnp.int32),
        scratch_types=[
            pltpu.VMEM((_CH2,), jnp.int32),
            pltpu.VMEM((_CH2,), jnp.int32),
            pltpu.VMEM((_CH2, D_MODEL // 2), jnp.int32),
            pltpu.VMEM((_CH2, D_MODEL // 2), jnp.int32),
            pltpu.SemaphoreType.DMA,
            pltpu.SemaphoreType.DMA,
        ],
    )
    def k(flat_hbm, dest_hbm, xs_hbm, didx0_v, didx1_v, rows0_v, rows1_v,
          sem0, sem1):
        c = lax.axis_index("c")
        s = lax.axis_index("s")
        wid = s * 2 + c
        half = wid // 16
        seg = wid % 16
        base = seg * (N_TOK // 16)
        didx = (didx0_v, didx1_v)
        rows = (rows0_v, rows1_v)
        sems = (sem0, sem1)
        hs = [None, None]

        def load(j, r):
            off = base + j * _CH2
            pltpu.sync_copy(flat_hbm.at[pl.ds(off, _CH2)], rows[r])
            pltpu.sync_copy(dest_hbm.at[pl.ds(half * N_TOK + off, _CH2)],
                            didx[r])

        load(0, 0)
        for j in range(nchunk):
            r = j % 2
            hs[r] = pltpu.async_copy(rows[r], xs_hbm.at[didx[r]], sems[r])
            if j + 1 < nchunk:
                if hs[1 - r] is not None:
                    hs[1 - r].wait()
                load(j + 1, 1 - r)
        hs[(nchunk - 1) % 2].wait()

    return k


def _gmm_body(be_map_ref, xs_ref, w_ref, b_ref, ys_ref):
    xlo, xhi = _unpack_pair(xs_ref[...])
    xbf = jnp.concatenate([xlo, xhi], axis=1).astype(jnp.bfloat16)
    acc = jnp.dot(xbf, w_ref[0], preferred_element_type=jnp.float32)
    acc = acc + b_ref[0]
    ys_ref[...] = _pack_pair(acc[:, :D_INNER // 2], acc[:, D_INNER // 2:])


def _gmm(block_expert, xs, We_bf, be):
    grid_spec = pltpu.PrefetchScalarGridSpec(
        num_scalar_prefetch=1,
        grid=(_NB,),
        in_specs=[
            pl.BlockSpec((_BM, D_MODEL // 2), lambda b, bm: (b, 0)),
            pl.BlockSpec((1, D_MODEL, D_INNER), lambda b, bm: (bm[b], 0, 0)),
            pl.BlockSpec((1, 1, D_INNER), lambda b, bm: (bm[b], 0, 0)),
        ],
        out_specs=pl.BlockSpec((_BM, D_INNER // 2), lambda b, bm: (b, 0)),
    )
    return pl.pallas_call(
        _gmm_body,
        grid_spec=grid_spec,
        out_shape=jax.ShapeDtypeStruct((_NSLOT, D_INNER // 2), jnp.int32),
        compiler_params=pltpu.CompilerParams(
            dimension_semantics=("arbitrary",),
        ),
    )(block_expert, xs, We_bf, be.reshape(N_EXPERTS, 1, D_INNER))


@functools.cache
def _make_pair_gather():
    mesh = plsc.VectorSubcoreMesh(core_axis_name="c", subcore_axis_name="s")
    per_w = N_TOK // _NW
    chunks = [24] * 10 + [16]
    cmax = max(chunks)

    @functools.partial(
        pl.kernel,
        mesh=mesh,
        out_type=[
            jax.ShapeDtypeStruct((N_TOK, D_INNER // 2), jnp.int32),
            jax.ShapeDtypeStruct((N_TOK, D_INNER // 2), jnp.int32),
        ],
        scratch_types=[
            pltpu.VMEM((cmax,), jnp.int32),
            pltpu.VMEM((cmax,), jnp.int32),
            pltpu.VMEM((cmax, D_INNER // 2), jnp.int32),
            pltpu.VMEM((cmax, D_INNER // 2), jnp.int32),
            pltpu.SemaphoreType.DMA,
            pltpu.SemaphoreType.DMA,
        ],
    )
    def k(ys_hbm, dest_hbm, y0_hbm, y1_hbm,
          idx0_v, idx1_v, b0_v, b1_v, sg, sw):
        c = lax.axis_index("c")
        s = lax.axis_index("s")
        wid = s * 2 + c
        base = wid * per_w
        off = 0
        hw = None
        for ch in chunks:
            o = base + off
            pltpu.sync_copy(dest_hbm.at[pl.ds(o, ch)],
                            idx0_v.at[pl.ds(0, ch)])
            pltpu.sync_copy(dest_hbm.at[pl.ds(N_TOK + o, ch)],
                            idx1_v.at[pl.ds(0, ch)])
            if hw is not None:
                hw[0].wait()
                hw[1].wait()
            cp0 = pltpu.async_copy(ys_hbm.at[idx0_v.at[pl.ds(0, ch)]],
                                   b0_v.at[pl.ds(0, ch)], sg)
            cp1 = pltpu.async_copy(ys_hbm.at[idx1_v.at[pl.ds(0, ch)]],
                                   b1_v.at[pl.ds(0, ch)], sg)
            cp0.wait()
            cp1.wait()
            hw = (
                pltpu.async_copy(b0_v.at[pl.ds(0, ch)],
                                 y0_hbm.at[pl.ds(o, ch)], sw),
                pltpu.async_copy(b1_v.at[pl.ds(0, ch)],
                                 y1_hbm.at[pl.ds(o, ch)], sw),
            )
            off += ch
        hw[0].wait()
        hw[1].wait()

    return k


def _combine_body(y0_ref, y1_ref, out_ref):
    lo0, hi0 = _unpack_pair(y0_ref[...])
    lo1, hi1 = _unpack_pair(y1_ref[...])
    out_ref[...] = jnp.concatenate(
        [lo0 + lo1, hi0 + hi1], axis=1) * (1.0 / TOP_K)


def _combine(y0, y1):
    bm = 512
    return pl.pallas_call(
        _combine_body,
        grid=(N_TOK // bm,),
        in_specs=[
            pl.BlockSpec((bm, D_INNER // 2), lambda i: (i, 0)),
            pl.BlockSpec((bm, D_INNER // 2), lambda i: (i, 0)),
        ],
        out_specs=pl.BlockSpec((bm, D_INNER), lambda i: (i, 0)),
        out_shape=jax.ShapeDtypeStruct((N_TOK, D_INNER), jnp.float32),
    )(y0, y1)


def kernel(sequences, We, be, Wg, bg):
    N, P, D = sequences.shape
    flat = sequences.reshape(-1, D)
    top2, var, flat_p, pos, counts = _gating(flat, Wg, bg)

    counts = counts.reshape(N_EXPERTS)
    padded = ((counts + _BM - 1) // _BM) * _BM
    cum = jnp.cumsum(padded)
    pstart = cum - padded
    dest = pstart[top2] + pos
    dest_flat = dest.T.reshape(-1)
    block_expert = jnp.clip(
        jnp.searchsorted(cum, jnp.arange(_NB, dtype=jnp.int32) * _BM,
                         side="right"),
        0, N_EXPERTS - 1).astype(jnp.int32)

    We_bf = We.astype(jnp.bfloat16)
    xs = _make_dispatch_scatter()(flat_p, dest_flat)
    ys = _gmm(block_expert, xs, We_bf, be)
    y0, y1 = _make_pair_gather()(ys, dest_flat)
    out = _combine(y0, y1)
    return (out.reshape(N, P, -1), var.reshape(N_EXPERTS))


# speedup vs baseline: 6.7979x; 1.0445x over previous
import functools

import jax
import jax.numpy as jnp
from jax import lax
from jax.experimental import pallas as pl
from jax.experimental.pallas import tpu as pltpu
from jax.experimental.pallas import tpu_sc as plsc

D_MODEL = 1024
D_INNER = 4096
N_EXPERTS = 8
TOP_K = 2

N_TOK = 8192
_GM = 512
_BM = 256
_NB = (N_TOK * TOP_K + N_EXPERTS * (_BM - 1) + _BM - 1) // _BM
_NSLOT = _NB * _BM
_NW = 32


def _cumsum0(a):
    n = a.shape[0]
    k = 1
    while k < n:
        a = a + jnp.pad(a, ((k, 0), (0, 0)))[:n]
        k *= 2
    return a


def _pack_pair(a, b):
    au = lax.bitcast_convert_type(a, jnp.uint32)
    bu = lax.bitcast_convert_type(b, jnp.uint32)
    ar = (au + 0x8000) >> 16
    br = (bu + 0x8000) & jnp.uint32(0xFFFF0000)
    return lax.bitcast_convert_type(ar | br, jnp.int32)


def _unpack_pair(w):
    wu = lax.bitcast_convert_type(w, jnp.uint32)
    lo = lax.bitcast_convert_type(wu << 16, jnp.float32)
    hi = lax.bitcast_convert_type(wu & jnp.uint32(0xFFFF0000), jnp.float32)
    return lo, hi


def _gating_body(x_ref, wg_ref, bg_ref,
                 top2_ref, var_ref, xp_ref, pos_ref, cnt_ref,
                 sum_ref, sq_ref, run_ref):
    i = pl.program_id(0)
    nt = pl.num_programs(0)
    x = x_ref[...]
    xp_ref[...] = _pack_pair(x[:, :D_MODEL // 2], x[:, D_MODEL // 2:])
    logits = jnp.dot(x, wg_ref[...], preferred_element_type=jnp.float32)
    logits = logits + bg_ref[...]
    m = jnp.max(logits, axis=1, keepdims=True)
    e = jnp.exp(logits - m)
    p = e / jnp.sum(e, axis=1, keepdims=True)
    col = jax.lax.broadcasted_iota(jnp.int32, p.shape, 1)
    p1 = jnp.max(p, axis=1, keepdims=True)
    i1 = jnp.min(jnp.where(p == p1, col, N_EXPERTS), axis=1, keepdims=True)
    pm = jnp.where(col == i1, -jnp.inf, p)
    p2 = jnp.max(pm, axis=1, keepdims=True)
    i2 = jnp.min(jnp.where(pm == p2, col, N_EXPERTS), axis=1, keepdims=True)
    top2_ref[...] = jnp.concatenate([i1, i2], axis=1)

    @pl.when(i == 0)
    def _():
        sum_ref[...] = jnp.zeros_like(sum_ref)
        sq_ref[...] = jnp.zeros_like(sq_ref)
        run_ref[...] = jnp.zeros_like(run_ref)

    sum_ref[...] += jnp.sum(p, axis=0, keepdims=True)
    sq_ref[...] += jnp.sum(p * p, axis=0, keepdims=True)

    oh = (col == i1).astype(jnp.int32) + (col == i2).astype(jnp.int32)
    cum = _cumsum0(oh)
    base = run_ref[...] + (cum - oh)
    pos0 = jnp.sum(jnp.where(col == i1, base, 0), axis=1, keepdims=True)
    pos1 = jnp.sum(jnp.where(col == i2, base, 0), axis=1, keepdims=True)
    pos_ref[...] = jnp.concatenate([pos0, pos1], axis=1)
    run_ref[...] += jnp.sum(oh, axis=0, keepdims=True)

    @pl.when(i == nt - 1)
    def _():
        n = nt * _GM
        mean = sum_ref[...] / n
        var_ref[...] = (sq_ref[...] - n * mean * mean) / (n - 1)
        cnt_ref[...] = run_ref[...]


def _gating(flat, Wg, bg):
    grid = (N_TOK // _GM,)
    return pl.pallas_call(
        _gating_body,
        grid=grid,
        in_specs=[
            pl.BlockSpec((_GM, D_MODEL), lambda i: (i, 0)),
            pl.BlockSpec((D_MODEL, N_EXPERTS), lambda i: (0, 0)),
            pl.BlockSpec((1, N_EXPERTS), lambda i: (0, 0)),
        ],
        out_specs=[
            pl.BlockSpec((_GM, TOP_K), lambda i: (i, 0)),
            pl.BlockSpec((1, N_EXPERTS), lambda i: (0, 0)),
            pl.BlockSpec((_GM, D_MODEL // 2), lambda i: (i, 0)),
            pl.BlockSpec((_GM, TOP_K), lambda i: (i, 0)),
            pl.BlockSpec((1, N_EXPERTS), lambda i: (0, 0)),
        ],
        out_shape=[
            jax.ShapeDtypeStruct((N_TOK, TOP_K), jnp.int32),
            jax.ShapeDtypeStruct((1, N_EXPERTS), jnp.float32),
            jax.ShapeDtypeStruct((N_TOK, D_MODEL // 2), jnp.int32),
            jax.ShapeDtypeStruct((N_TOK, TOP_K), jnp.int32),
            jax.ShapeDtypeStruct((1, N_EXPERTS), jnp.int32),
        ],
        scratch_shapes=[
            pltpu.VMEM((1, N_EXPERTS), jnp.float32),
            pltpu.VMEM((1, N_EXPERTS), jnp.float32),
            pltpu.VMEM((1, N_EXPERTS), jnp.int32),
        ],
    )(flat, Wg, bg.reshape(1, N_EXPERTS))


_CH2 = 64
_CH4 = 8


@functools.cache
def _make_dispatch_scatter():
    mesh = plsc.VectorSubcoreMesh(core_axis_name="c", subcore_axis_name="s")

    nchunk = N_TOK // 16 // _CH2

    @functools.partial(
        pl.kernel,
        mesh=mesh,
        out_type=jax.ShapeDtypeStruct((_NSLOT, D_MODEL // 2), jnp.int32),
        scratch_types=[
            pltpu.VMEM((_CH2,), jnp.int32),
            pltpu.VMEM((_CH2,), jnp.int32),
            pltpu.VMEM((_CH2, D_MODEL // 2), jnp.int32),
            pltpu.VMEM((_CH2, D_MODEL // 2), jnp.int32),
            pltpu.SemaphoreType.DMA,
            pltpu.SemaphoreType.DMA,
        ],
    )
    def k(flat_hbm, dest_hbm, xs_hbm, didx0_v, didx1_v, rows0_v, rows1_v,
          sem0, sem1):
        c = lax.axis_index("c")
        s = lax.axis_index("s")
        wid = s * 2 + c
        half = wid // 16
        seg = wid % 16
        base = seg * (N_TOK // 16)
        didx = (didx0_v, didx1_v)
        rows = (rows0_v, rows1_v)
        sems = (sem0, sem1)
        hs = [None, None]

        def load(j, r):
            off = base + j * _CH2
            pltpu.sync_copy(flat_hbm.at[pl.ds(off, _CH2)], rows[r])
            pltpu.sync_copy(dest_hbm.at[pl.ds(half * N_TOK + off, _CH2)],
                            didx[r])

        load(0, 0)
        for j in range(nchunk):
            r = j % 2
            hs[r] = pltpu.async_copy(rows[r], xs_hbm.at[didx[r]], sems[r])
            if j + 1 < nchunk:
                if hs[1 - r] is not None:
                    hs[1 - r].wait()
                load(j + 1, 1 - r)
        hs[(nchunk - 1) % 2].wait()

    return k


def _wcast_body(w_ref, o_ref):
    o_ref[...] = w_ref[...].astype(jnp.bfloat16)


def _wcast(We):
    return pl.pallas_call(
        _wcast_body,
        grid=(N_EXPERTS * 2,),
        in_specs=[pl.BlockSpec((1, D_MODEL // 2, D_INNER),
                               lambda i: (i // 2, i % 2, 0))],
        out_specs=pl.BlockSpec((1, D_MODEL // 2, D_INNER),
                               lambda i: (i // 2, i % 2, 0)),
        out_shape=jax.ShapeDtypeStruct((N_EXPERTS, D_MODEL, D_INNER),
                                       jnp.bfloat16),
    )(We)


def _gmm_body(be_map_ref, xs_ref, w_ref, b_ref, ys_ref):
    xlo, xhi = _unpack_pair(xs_ref[...])
    xbf = jnp.concatenate([xlo, xhi], axis=1).astype(jnp.bfloat16)
    acc = jnp.dot(xbf, w_ref[0], preferred_element_type=jnp.float32)
    acc = acc + b_ref[0]
    ys_ref[...] = _pack_pair(acc[:, :D_INNER // 2], acc[:, D_INNER // 2:])


def _gmm(block_expert, xs, We_bf, be):
    grid_spec = pltpu.PrefetchScalarGridSpec(
        num_scalar_prefetch=1,
        grid=(_NB,),
        in_specs=[
            pl.BlockSpec((_BM, D_MODEL // 2), lambda b, bm: (b, 0)),
            pl.BlockSpec((1, D_MODEL, D_INNER), lambda b, bm: (bm[b], 0, 0)),
            pl.BlockSpec((1, 1, D_INNER), lambda b, bm: (bm[b], 0, 0)),
        ],
        out_specs=pl.BlockSpec((_BM, D_INNER // 2), lambda b, bm: (b, 0)),
    )
    return pl.pallas_call(
        _gmm_body,
        grid_spec=grid_spec,
        out_shape=jax.ShapeDtypeStruct((_NSLOT, D_INNER // 2), jnp.int32),
        compiler_params=pltpu.CompilerParams(
            dimension_semantics=("arbitrary",),
        ),
    )(block_expert, xs, We_bf, be.reshape(N_EXPERTS, 1, D_INNER))


@functools.cache
def _make_pair_gather():
    mesh = plsc.VectorSubcoreMesh(core_axis_name="c", subcore_axis_name="s")
    per_w = N_TOK // _NW
    chunks = [24] * 10 + [16]
    cmax = max(chunks)

    @functools.partial(
        pl.kernel,
        mesh=mesh,
        out_type=[
            jax.ShapeDtypeStruct((N_TOK, D_INNER // 2), jnp.int32),
            jax.ShapeDtypeStruct((N_TOK, D_INNER // 2), jnp.int32),
        ],
        scratch_types=[
            pltpu.VMEM((cmax,), jnp.int32),
            pltpu.VMEM((cmax,), jnp.int32),
            pltpu.VMEM((cmax, D_INNER // 2), jnp.int32),
            pltpu.VMEM((cmax, D_INNER // 2), jnp.int32),
            pltpu.SemaphoreType.DMA,
            pltpu.SemaphoreType.DMA,
        ],
    )
    def k(ys_hbm, dest_hbm, y0_hbm, y1_hbm,
          idx0_v, idx1_v, b0_v, b1_v, sg, sw):
        c = lax.axis_index("c")
        s = lax.axis_index("s")
        wid = s * 2 + c
        base = wid * per_w
        off = 0
        hw = None
        for ch in chunks:
            o = base + off
            pltpu.sync_copy(dest_hbm.at[pl.ds(o, ch)],
                            idx0_v.at[pl.ds(0, ch)])
            pltpu.sync_copy(dest_hbm.at[pl.ds(N_TOK + o, ch)],
                            idx1_v.at[pl.ds(0, ch)])
            if hw is not None:
                hw[0].wait()
                hw[1].wait()
            cp0 = pltpu.async_copy(ys_hbm.at[idx0_v.at[pl.ds(0, ch)]],
                                   b0_v.at[pl.ds(0, ch)], sg)
            cp1 = pltpu.async_copy(ys_hbm.at[idx1_v.at[pl.ds(0, ch)]],
                                   b1_v.at[pl.ds(0, ch)], sg)
            cp0.wait()
            cp1.wait()
            hw = (
                pltpu.async_copy(b0_v.at[pl.ds(0, ch)],
                                 y0_hbm.at[pl.ds(o, ch)], sw),
                pltpu.async_copy(b1_v.at[pl.ds(0, ch)],
                                 y1_hbm.at[pl.ds(o, ch)], sw),
            )
            off += ch
        hw[0].wait()
        hw[1].wait()

    return k


def _combine_body(y0_ref, y1_ref, out_ref):
    lo0, hi0 = _unpack_pair(y0_ref[...])
    lo1, hi1 = _unpack_pair(y1_ref[...])
    out_ref[...] = jnp.concatenate(
        [lo0 + lo1, hi0 + hi1], axis=1) * (1.0 / TOP_K)


def _combine(y0, y1):
    bm = 512
    return pl.pallas_call(
        _combine_body,
        grid=(N_TOK // bm,),
        in_specs=[
            pl.BlockSpec((bm, D_INNER // 2), lambda i: (i, 0)),
            pl.BlockSpec((bm, D_INNER // 2), lambda i: (i, 0)),
        ],
        out_specs=pl.BlockSpec((bm, D_INNER), lambda i: (i, 0)),
        out_shape=jax.ShapeDtypeStruct((N_TOK, D_INNER), jnp.float32),
    )(y0, y1)


def kernel(sequences, We, be, Wg, bg):
    N, P, D = sequences.shape
    flat = sequences.reshape(-1, D)
    top2, var, flat_p, pos, counts = _gating(flat, Wg, bg)

    counts = counts.reshape(N_EXPERTS)
    padded = ((counts + _BM - 1) // _BM) * _BM
    cum = jnp.cumsum(padded)
    pstart = cum - padded
    dest = pstart[top2] + pos
    dest_flat = dest.T.reshape(-1)
    block_expert = jnp.clip(
        jnp.searchsorted(cum, jnp.arange(_NB, dtype=jnp.int32) * _BM,
                         side="right"),
        0, N_EXPERTS - 1).astype(jnp.int32)

    We_bf = _wcast(We)
    xs = _make_dispatch_scatter()(flat_p, dest_flat)
    ys = _gmm(block_expert, xs, We_bf, be)
    y0, y1 = _make_pair_gather()(ys, dest_flat)
    out = _combine(y0, y1)
    return (out.reshape(N, P, -1), var.reshape(N_EXPERTS))


# speedup vs baseline: 7.0255x; 1.0335x over previous
import functools

import jax
import jax.numpy as jnp
from jax import lax
from jax.experimental import pallas as pl
from jax.experimental.pallas import tpu as pltpu
from jax.experimental.pallas import tpu_sc as plsc

D_MODEL = 1024
D_INNER = 4096
N_EXPERTS = 8
TOP_K = 2

N_TOK = 8192
_GM = 512
_BM = 256
_NB = (N_TOK * TOP_K + N_EXPERTS * (_BM - 1) + _BM - 1) // _BM
_NSLOT = _NB * _BM
_NW = 32


def _cumsum0(a):
    n = a.shape[0]
    k = 1
    while k < n:
        a = a + jnp.pad(a, ((k, 0), (0, 0)))[:n]
        k *= 2
    return a


def _pack_pair(a, b):
    au = lax.bitcast_convert_type(a, jnp.uint32)
    bu = lax.bitcast_convert_type(b, jnp.uint32)
    ar = (au + 0x8000) >> 16
    br = (bu + 0x8000) & jnp.uint32(0xFFFF0000)
    return lax.bitcast_convert_type(ar | br, jnp.int32)


def _unpack_pair(w):
    wu = lax.bitcast_convert_type(w, jnp.uint32)
    lo = lax.bitcast_convert_type(wu << 16, jnp.float32)
    hi = lax.bitcast_convert_type(wu & jnp.uint32(0xFFFF0000), jnp.float32)
    return lo, hi


def _gating_body(x_ref, wg_ref, bg_ref,
                 top2_ref, var_ref, xp_ref, pos_ref, cnt_ref,
                 sum_ref, sq_ref, run_ref):
    i = pl.program_id(0)
    nt = pl.num_programs(0)
    x = x_ref[...]
    xp_ref[...] = _pack_pair(x[:, :D_MODEL // 2], x[:, D_MODEL // 2:])
    logits = jnp.dot(x, wg_ref[...], preferred_element_type=jnp.float32)
    logits = logits + bg_ref[...]
    m = jnp.max(logits, axis=1, keepdims=True)
    e = jnp.exp(logits - m)
    p = e / jnp.sum(e, axis=1, keepdims=True)
    col = jax.lax.broadcasted_iota(jnp.int32, p.shape, 1)
    p1 = jnp.max(p, axis=1, keepdims=True)
    i1 = jnp.min(jnp.where(p == p1, col, N_EXPERTS), axis=1, keepdims=True)
    pm = jnp.where(col == i1, -jnp.inf, p)
    p2 = jnp.max(pm, axis=1, keepdims=True)
    i2 = jnp.min(jnp.where(pm == p2, col, N_EXPERTS), axis=1, keepdims=True)
    top2_ref[...] = jnp.concatenate([i1, i2], axis=1)

    @pl.when(i == 0)
    def _():
        sum_ref[...] = jnp.zeros_like(sum_ref)
        sq_ref[...] = jnp.zeros_like(sq_ref)
        run_ref[...] = jnp.zeros_like(run_ref)

    sum_ref[...] += jnp.sum(p, axis=0, keepdims=True)
    sq_ref[...] += jnp.sum(p * p, axis=0, keepdims=True)

    oh = (col == i1).astype(jnp.int32) + (col == i2).astype(jnp.int32)
    cum = _cumsum0(oh)
    base = run_ref[...] + (cum - oh)
    pos0 = jnp.sum(jnp.where(col == i1, base, 0), axis=1, keepdims=True)
    pos1 = jnp.sum(jnp.where(col == i2, base, 0), axis=1, keepdims=True)
    pos_ref[...] = jnp.concatenate([pos0, pos1], axis=1)
    run_ref[...] += jnp.sum(oh, axis=0, keepdims=True)

    @pl.when(i == nt - 1)
    def _():
        n = nt * _GM
        mean = sum_ref[...] / n
        var_ref[...] = (sq_ref[...] - n * mean * mean) / (n - 1)
        cnt_ref[...] = run_ref[...]


def _gating(flat, Wg, bg):
    grid = (N_TOK // _GM,)
    return pl.pallas_call(
        _gating_body,
        grid=grid,
        in_specs=[
            pl.BlockSpec((_GM, D_MODEL), lambda i: (i, 0)),
            pl.BlockSpec((D_MODEL, N_EXPERTS), lambda i: (0, 0)),
            pl.BlockSpec((1, N_EXPERTS), lambda i: (0, 0)),
        ],
        out_specs=[
            pl.BlockSpec((_GM, TOP_K), lambda i: (i, 0)),
            pl.BlockSpec((1, N_EXPERTS), lambda i: (0, 0)),
            pl.BlockSpec((_GM, D_MODEL // 2), lambda i: (i, 0)),
            pl.BlockSpec((_GM, TOP_K), lambda i: (i, 0)),
            pl.BlockSpec((1, N_EXPERTS), lambda i: (0, 0)),
        ],
        out_shape=[
            jax.ShapeDtypeStruct((N_TOK, TOP_K), jnp.int32),
            jax.ShapeDtypeStruct((1, N_EXPERTS), jnp.float32),
            jax.ShapeDtypeStruct((N_TOK, D_MODEL // 2), jnp.int32),
            jax.ShapeDtypeStruct((N_TOK, TOP_K), jnp.int32),
            jax.ShapeDtypeStruct((1, N_EXPERTS), jnp.int32),
        ],
        scratch_shapes=[
            pltpu.VMEM((1, N_EXPERTS), jnp.float32),
            pltpu.VMEM((1, N_EXPERTS), jnp.float32),
            pltpu.VMEM((1, N_EXPERTS), jnp.int32),
        ],
    )(flat, Wg, bg.reshape(1, N_EXPERTS))


_CH2 = 64
_CH4 = 8


@functools.cache
def _make_dispatch_scatter():
    mesh = plsc.VectorSubcoreMesh(core_axis_name="c", subcore_axis_name="s")

    nchunk = N_TOK // 16 // _CH2

    @functools.partial(
        pl.kernel,
        mesh=mesh,
        out_type=jax.ShapeDtypeStruct((_NSLOT, D_MODEL // 2), jnp.int32),
        scratch_types=[
            pltpu.VMEM((_CH2,), jnp.int32),
            pltpu.VMEM((_CH2,), jnp.int32),
            pltpu.VMEM((_CH2, D_MODEL // 2), jnp.int32),
            pltpu.VMEM((_CH2, D_MODEL // 2), jnp.int32),
            pltpu.SemaphoreType.DMA,
            pltpu.SemaphoreType.DMA,
        ],
    )
    def k(flat_hbm, dest_hbm, xs_hbm, didx0_v, didx1_v, rows0_v, rows1_v,
          sem0, sem1):
        c = lax.axis_index("c")
        s = lax.axis_index("s")
        wid = s * 2 + c
        half = wid // 16
        seg = wid % 16
        base = seg * (N_TOK // 16)
        didx = (didx0_v, didx1_v)
        rows = (rows0_v, rows1_v)
        sems = (sem0, sem1)
        hs = [None, None]

        def load(j, r):
            off = base + j * _CH2
            pltpu.sync_copy(flat_hbm.at[pl.ds(off, _CH2)], rows[r])
            pltpu.sync_copy(dest_hbm.at[pl.ds(half * N_TOK + off, _CH2)],
                            didx[r])

        load(0, 0)
        for j in range(nchunk):
            r = j % 2
            hs[r] = pltpu.async_copy(rows[r], xs_hbm.at[didx[r]], sems[r])
            if j + 1 < nchunk:
                if hs[1 - r] is not None:
                    hs[1 - r].wait()
                load(j + 1, 1 - r)
        hs[(nchunk - 1) % 2].wait()
        if hs[nchunk % 2] is not None:
            hs[nchunk % 2].wait()

    return k


def _wcast_body(w_ref, o_ref):
    o_ref[...] = w_ref[...].astype(jnp.bfloat16)


def _wcast(We):
    return pl.pallas_call(
        _wcast_body,
        grid=(N_EXPERTS * 2,),
        in_specs=[pl.BlockSpec((1, D_MODEL // 2, D_INNER),
                               lambda i: (i // 2, i % 2, 0))],
        out_specs=pl.BlockSpec((1, D_MODEL // 2, D_INNER),
                               lambda i: (i // 2, i % 2, 0)),
        out_shape=jax.ShapeDtypeStruct((N_EXPERTS, D_MODEL, D_INNER),
                                       jnp.bfloat16),
    )(We)


def _gmm_body(be_map_ref, xs_ref, w_ref, b_ref, ys_ref):
    xlo, xhi = _unpack_pair(xs_ref[...])
    xbf = jnp.concatenate([xlo, xhi], axis=1).astype(jnp.bfloat16)
    acc = jnp.dot(xbf, w_ref[0], preferred_element_type=jnp.float32)
    acc = acc + b_ref[0]
    ys_ref[...] = _pack_pair(acc[:, :D_INNER // 2], acc[:, D_INNER // 2:])


def _gmm(block_expert, xs, We_bf, be):
    grid_spec = pltpu.PrefetchScalarGridSpec(
        num_scalar_prefetch=1,
        grid=(_NB,),
        in_specs=[
            pl.BlockSpec((_BM, D_MODEL // 2), lambda b, bm: (b, 0)),
            pl.BlockSpec((1, D_MODEL, D_INNER), lambda b, bm: (bm[b], 0, 0)),
            pl.BlockSpec((1, 1, D_INNER), lambda b, bm: (bm[b], 0, 0)),
        ],
        out_specs=pl.BlockSpec((_BM, D_INNER // 2), lambda b, bm: (b, 0)),
    )
    return pl.pallas_call(
        _gmm_body,
        grid_spec=grid_spec,
        out_shape=jax.ShapeDtypeStruct((_NSLOT, D_INNER // 2), jnp.int32),
        compiler_params=pltpu.CompilerParams(
            dimension_semantics=("arbitrary",),
        ),
    )(block_expert, xs, We_bf, be.reshape(N_EXPERTS, 1, D_INNER))


@functools.cache
def _make_pair_gather():
    mesh = plsc.VectorSubcoreMesh(core_axis_name="c", subcore_axis_name="s")
    per_w = N_TOK // _NW
    chunks = [24] * 10 + [16]
    cmax = max(chunks)

    @functools.partial(
        pl.kernel,
        mesh=mesh,
        out_type=[
            jax.ShapeDtypeStruct((N_TOK, D_INNER // 2), jnp.int32),
            jax.ShapeDtypeStruct((N_TOK, D_INNER // 2), jnp.int32),
        ],
        scratch_types=[
            pltpu.VMEM((cmax,), jnp.int32),
            pltpu.VMEM((cmax,), jnp.int32),
            pltpu.VMEM((cmax,), jnp.int32),
            pltpu.VMEM((cmax,), jnp.int32),
            pltpu.VMEM((cmax, D_INNER // 2), jnp.int32),
            pltpu.VMEM((cmax, D_INNER // 2), jnp.int32),
            pltpu.SemaphoreType.DMA,
            pltpu.SemaphoreType.DMA,
        ],
    )
    def k(ys_hbm, dest_hbm, y0_hbm, y1_hbm,
          idx0_v, idx0b_v, idx1_v, idx1b_v, b0_v, b1_v, sg, sw):
        c = lax.axis_index("c")
        s = lax.axis_index("s")
        wid = s * 2 + c
        base = wid * per_w
        off = 0
        hw = None
        for ch in chunks:
            o = base + off
            pltpu.sync_copy(dest_hbm.at[pl.ds(o, ch)],
                            idx0_v.at[pl.ds(0, ch)])
            pltpu.sync_copy(dest_hbm.at[pl.ds(N_TOK + o, ch)],
                            idx1_v.at[pl.ds(0, ch)])
            if hw is not None:
                hw[0].wait()
                hw[1].wait()
            cp0 = pltpu.async_copy(ys_hbm.at[idx0_v.at[pl.ds(0, ch)]],
                                   b0_v.at[pl.ds(0, ch)], sg)
            cp1 = pltpu.async_copy(ys_hbm.at[idx1_v.at[pl.ds(0, ch)]],
                                   b1_v.at[pl.ds(0, ch)], sg)
            cp0.wait()
            cp1.wait()
            hw = (
                pltpu.async_copy(b0_v.at[pl.ds(0, ch)],
                                 y0_hbm.at[pl.ds(o, ch)], sw),
                pltpu.async_copy(b1_v.at[pl.ds(0, ch)],
                                 y1_hbm.at[pl.ds(o, ch)], sw),
            )
            off += ch
        hw[0].wait()
        hw[1].wait()

    return k


def _combine_body(y0_ref, y1_ref, out_ref):
    lo0, hi0 = _unpack_pair(y0_ref[...])
    lo1, hi1 = _unpack_pair(y1_ref[...])
    out_ref[...] = jnp.concatenate(
        [lo0 + lo1, hi0 + hi1], axis=1) * (1.0 / TOP_K)


def _combine(y0, y1):
    bm = 512
    return pl.pallas_call(
        _combine_body,
        grid=(N_TOK // bm,),
        in_specs=[
            pl.BlockSpec((bm, D_INNER // 2), lambda i: (i, 0)),
            pl.BlockSpec((bm, D_INNER // 2), lambda i: (i, 0)),
        ],
        out_specs=pl.BlockSpec((bm, D_INNER), lambda i: (i, 0)),
        out_shape=jax.ShapeDtypeStruct((N_TOK, D_INNER), jnp.float32),
    )(y0, y1)


def kernel(sequences, We, be, Wg, bg):
    N, P, D = sequences.shape
    flat = sequences.reshape(-1, D)
    top2, var, flat_p, pos, counts = _gating(flat, Wg, bg)

    counts = counts.reshape(N_EXPERTS)
    padded = ((counts + _BM - 1) // _BM) * _BM
    cum = jnp.cumsum(padded)
    pstart = cum - padded
    dest = pstart[top2] + pos
    dest_flat = dest.T.reshape(-1)
    starts = jnp.arange(_NB, dtype=jnp.int32) * _BM
    block_expert = jnp.minimum(
        jnp.sum((starts[:, None] >= cum[None, :]).astype(jnp.int32), axis=1),
        N_EXPERTS - 1).astype(jnp.int32)

    We_bf = _wcast(We)
    xs = _make_dispatch_scatter()(flat_p, dest_flat)
    ys = _gmm(block_expert, xs, We_bf, be)
    y0, y1 = _make_pair_gather()(ys, dest_flat)
    out = _combine(y0, y1)
    return (out.reshape(N, P, -1), var.reshape(N_EXPERTS))


# speedup vs baseline: 7.1639x; 1.0197x over previous
import functools

import jax
import jax.numpy as jnp
from jax import lax
from jax.experimental import pallas as pl
from jax.experimental.pallas import tpu as pltpu
from jax.experimental.pallas import tpu_sc as plsc

D_MODEL = 1024
D_INNER = 4096
N_EXPERTS = 8
TOP_K = 2

N_TOK = 8192
_GM = 512
_BM = 512
_NB = (N_TOK * TOP_K + N_EXPERTS * (_BM - 1) + _BM - 1) // _BM
_NSLOT = _NB * _BM
_NW = 32


def _cumsum0(a):
    n = a.shape[0]
    k = 1
    while k < n:
        a = a + jnp.pad(a, ((k, 0), (0, 0)))[:n]
        k *= 2
    return a


def _pack_pair(a, b):
    au = lax.bitcast_convert_type(a, jnp.uint32)
    bu = lax.bitcast_convert_type(b, jnp.uint32)
    ar = (au + 0x8000) >> 16
    br = (bu + 0x8000) & jnp.uint32(0xFFFF0000)
    return lax.bitcast_convert_type(ar | br, jnp.int32)


def _unpack_pair(w):
    wu = lax.bitcast_convert_type(w, jnp.uint32)
    lo = lax.bitcast_convert_type(wu << 16, jnp.float32)
    hi = lax.bitcast_convert_type(wu & jnp.uint32(0xFFFF0000), jnp.float32)
    return lo, hi


def _gating_body(x_ref, wg_ref, bg_ref,
                 top2_ref, var_ref, xp_ref, pos_ref, cnt_ref,
                 sum_ref, sq_ref, run_ref):
    i = pl.program_id(0)
    nt = pl.num_programs(0)
    x = x_ref[...]
    xp_ref[...] = _pack_pair(x[:, :D_MODEL // 2], x[:, D_MODEL // 2:])
    logits = jnp.dot(x, wg_ref[...], preferred_element_type=jnp.float32)
    logits = logits + bg_ref[...]
    m = jnp.max(logits, axis=1, keepdims=True)
    e = jnp.exp(logits - m)
    p = e / jnp.sum(e, axis=1, keepdims=True)
    col = jax.lax.broadcasted_iota(jnp.int32, p.shape, 1)
    p1 = jnp.max(p, axis=1, keepdims=True)
    i1 = jnp.min(jnp.where(p == p1, col, N_EXPERTS), axis=1, keepdims=True)
    pm = jnp.where(col == i1, -jnp.inf, p)
    p2 = jnp.max(pm, axis=1, keepdims=True)
    i2 = jnp.min(jnp.where(pm == p2, col, N_EXPERTS), axis=1, keepdims=True)
    top2_ref[...] = jnp.concatenate([i1, i2], axis=1)

    @pl.when(i == 0)
    def _():
        sum_ref[...] = jnp.zeros_like(sum_ref)
        sq_ref[...] = jnp.zeros_like(sq_ref)
        run_ref[...] = jnp.zeros_like(run_ref)

    sum_ref[...] += jnp.sum(p, axis=0, keepdims=True)
    sq_ref[...] += jnp.sum(p * p, axis=0, keepdims=True)

    oh = (col == i1).astype(jnp.int32) + (col == i2).astype(jnp.int32)
    cum = _cumsum0(oh)
    base = run_ref[...] + (cum - oh)
    pos0 = jnp.sum(jnp.where(col == i1, base, 0), axis=1, keepdims=True)
    pos1 = jnp.sum(jnp.where(col == i2, base, 0), axis=1, keepdims=True)
    pos_ref[...] = jnp.concatenate([pos0, pos1], axis=1)
    run_ref[...] += jnp.sum(oh, axis=0, keepdims=True)

    @pl.when(i == nt - 1)
    def _():
        n = nt * _GM
        mean = sum_ref[...] / n
        var_ref[...] = (sq_ref[...] - n * mean * mean) / (n - 1)
        cnt_ref[...] = run_ref[...]


def _gating(flat, Wg, bg):
    grid = (N_TOK // _GM,)
    return pl.pallas_call(
        _gating_body,
        grid=grid,
        in_specs=[
            pl.BlockSpec((_GM, D_MODEL), lambda i: (i, 0)),
            pl.BlockSpec((D_MODEL, N_EXPERTS), lambda i: (0, 0)),
            pl.BlockSpec((1, N_EXPERTS), lambda i: (0, 0)),
        ],
        out_specs=[
            pl.BlockSpec((_GM, TOP_K), lambda i: (i, 0)),
            pl.BlockSpec((1, N_EXPERTS), lambda i: (0, 0)),
            pl.BlockSpec((_GM, D_MODEL // 2), lambda i: (i, 0)),
            pl.BlockSpec((_GM, TOP_K), lambda i: (i, 0)),
            pl.BlockSpec((1, N_EXPERTS), lambda i: (0, 0)),
        ],
        out_shape=[
            jax.ShapeDtypeStruct((N_TOK, TOP_K), jnp.int32),
            jax.ShapeDtypeStruct((1, N_EXPERTS), jnp.float32),
            jax.ShapeDtypeStruct((N_TOK, D_MODEL // 2), jnp.int32),
            jax.ShapeDtypeStruct((N_TOK, TOP_K), jnp.int32),
            jax.ShapeDtypeStruct((1, N_EXPERTS), jnp.int32),
        ],
        scratch_shapes=[
            pltpu.VMEM((1, N_EXPERTS), jnp.float32),
            pltpu.VMEM((1, N_EXPERTS), jnp.float32),
            pltpu.VMEM((1, N_EXPERTS), jnp.int32),
        ],
    )(flat, Wg, bg.reshape(1, N_EXPERTS))


_CH2 = 64
_CH4 = 8


@functools.cache
def _make_dispatch_scatter():
    mesh = plsc.VectorSubcoreMesh(core_axis_name="c", subcore_axis_name="s")

    nchunk = N_TOK // 16 // _CH2

    @functools.partial(
        pl.kernel,
        mesh=mesh,
        out_type=jax.ShapeDtypeStruct((_NSLOT, D_MODEL // 2), jnp.int32),
        scratch_types=[
            pltpu.VMEM((_CH2,), jnp.int32),
            pltpu.VMEM((_CH2,), jnp.int32),
            pltpu.VMEM((_CH2, D_MODEL // 2), jnp.int32),
            pltpu.VMEM((_CH2, D_MODEL // 2), jnp.int32),
            pltpu.SemaphoreType.DMA,
            pltpu.SemaphoreType.DMA,
        ],
    )
    def k(flat_hbm, dest_hbm, xs_hbm, didx0_v, didx1_v, rows0_v, rows1_v,
          sem0, sem1):
        c = lax.axis_index("c")
        s = lax.axis_index("s")
        wid = s * 2 + c
        half = wid // 16
        seg = wid % 16
        base = seg * (N_TOK // 16)
        didx = (didx0_v, didx1_v)
        rows = (rows0_v, rows1_v)
        sems = (sem0, sem1)
        hs = [None, None]

        def load(j, r):
            off = base + j * _CH2
            pltpu.sync_copy(flat_hbm.at[pl.ds(off, _CH2)], rows[r])
            pltpu.sync_copy(dest_hbm.at[pl.ds(half * N_TOK + off, _CH2)],
                            didx[r])

        load(0, 0)
        for j in range(nchunk):
            r = j % 2
            hs[r] = pltpu.async_copy(rows[r], xs_hbm.at[didx[r]], sems[r])
            if j + 1 < nchunk:
                if hs[1 - r] is not None:
                    hs[1 - r].wait()
                load(j + 1, 1 - r)
        hs[(nchunk - 1) % 2].wait()
        if hs[nchunk % 2] is not None:
            hs[nchunk % 2].wait()

    return k


def _wcast_body(w_ref, o_ref):
    o_ref[...] = w_ref[...].astype(jnp.bfloat16)


def _wcast(We):
    return pl.pallas_call(
        _wcast_body,
        grid=(N_EXPERTS * 2,),
        in_specs=[pl.BlockSpec((1, D_MODEL // 2, D_INNER),
                               lambda i: (i // 2, i % 2, 0))],
        out_specs=pl.BlockSpec((1, D_MODEL // 2, D_INNER),
                               lambda i: (i // 2, i % 2, 0)),
        out_shape=jax.ShapeDtypeStruct((N_EXPERTS, D_MODEL, D_INNER),
                                       jnp.bfloat16),
    )(We)


def _gmm_body(be_map_ref, xs_ref, w_ref, b_ref, ys_ref):
    xlo, xhi = _unpack_pair(xs_ref[...])
    xbf = jnp.concatenate([xlo, xhi], axis=1).astype(jnp.bfloat16)
    acc = jnp.dot(xbf, w_ref[0], preferred_element_type=jnp.float32)
    acc = acc + b_ref[0]
    ys_ref[...] = _pack_pair(acc[:, :D_INNER // 2], acc[:, D_INNER // 2:])


def _gmm(block_expert, xs, We_bf, be):
    grid_spec = pltpu.PrefetchScalarGridSpec(
        num_scalar_prefetch=1,
        grid=(_NB,),
        in_specs=[
            pl.BlockSpec((_BM, D_MODEL // 2), lambda b, bm: (b, 0)),
            pl.BlockSpec((1, D_MODEL, D_INNER), lambda b, bm: (bm[b], 0, 0)),
            pl.BlockSpec((1, 1, D_INNER), lambda b, bm: (bm[b], 0, 0)),
        ],
        out_specs=pl.BlockSpec((_BM, D_INNER // 2), lambda b, bm: (b, 0)),
    )
    return pl.pallas_call(
        _gmm_body,
        grid_spec=grid_spec,
        out_shape=jax.ShapeDtypeStruct((_NSLOT, D_INNER // 2), jnp.int32),
        compiler_params=pltpu.CompilerParams(
            dimension_semantics=("arbitrary",),
        ),
    )(block_expert, xs, We_bf, be.reshape(N_EXPERTS, 1, D_INNER))


@functools.cache
def _make_pair_gather():
    mesh = plsc.VectorSubcoreMesh(core_axis_name="c", subcore_axis_name="s")
    per_w = N_TOK // _NW
    chunks = [24] * 10 + [16]
    cmax = max(chunks)

    @functools.partial(
        pl.kernel,
        mesh=mesh,
        out_type=[
            jax.ShapeDtypeStruct((N_TOK, D_INNER // 2), jnp.int32),
            jax.ShapeDtypeStruct((N_TOK, D_INNER // 2), jnp.int32),
        ],
        scratch_types=[
            pltpu.VMEM((cmax,), jnp.int32),
            pltpu.VMEM((cmax,), jnp.int32),
            pltpu.VMEM((cmax,), jnp.int32),
            pltpu.VMEM((cmax,), jnp.int32),
            pltpu.VMEM((cmax, D_INNER // 2), jnp.int32),
            pltpu.VMEM((cmax, D_INNER // 2), jnp.int32),
            pltpu.SemaphoreType.DMA,
            pltpu.SemaphoreType.DMA,
        ],
    )
    def k(ys_hbm, dest_hbm, y0_hbm, y1_hbm,
          idx0_v, idx0b_v, idx1_v, idx1b_v, b0_v, b1_v, sg, sw):
        c = lax.axis_index("c")
        s = lax.axis_index("s")
        wid = s * 2 + c
        base = wid * per_w
        off = 0
        hw = None
        for ch in chunks:
            o = base + off
            pltpu.sync_copy(dest_hbm.at[pl.ds(o, ch)],
                            idx0_v.at[pl.ds(0, ch)])
            pltpu.sync_copy(dest_hbm.at[pl.ds(N_TOK + o, ch)],
                            idx1_v.at[pl.ds(0, ch)])
            if hw is not None:
                hw[0].wait()
                hw[1].wait()
            cp0 = pltpu.async_copy(ys_hbm.at[idx0_v.at[pl.ds(0, ch)]],
                                   b0_v.at[pl.ds(0, ch)], sg)
            cp1 = pltpu.async_copy(ys_hbm.at[idx1_v.at[pl.ds(0, ch)]],
                                   b1_v.at[pl.ds(0, ch)], sg)
            cp0.wait()
            cp1.wait()
            hw = (
                pltpu.async_copy(b0_v.at[pl.ds(0, ch)],
                                 y0_hbm.at[pl.ds(o, ch)], sw),
                pltpu.async_copy(b1_v.at[pl.ds(0, ch)],
                                 y1_hbm.at[pl.ds(o, ch)], sw),
            )
            off += ch
        hw[0].wait()
        hw[1].wait()

    return k


def _combine_body(y0_ref, y1_ref, out_ref):
    lo0, hi0 = _unpack_pair(y0_ref[...])
    lo1, hi1 = _unpack_pair(y1_ref[...])
    out_ref[...] = jnp.concatenate(
        [lo0 + lo1, hi0 + hi1], axis=1) * (1.0 / TOP_K)


def _combine(y0, y1):
    bm = 512
    return pl.pallas_call(
        _combine_body,
        grid=(N_TOK // bm,),
        in_specs=[
            pl.BlockSpec((bm, D_INNER // 2), lambda i: (i, 0)),
            pl.BlockSpec((bm, D_INNER // 2), lambda i: (i, 0)),
        ],
        out_specs=pl.BlockSpec((bm, D_INNER), lambda i: (i, 0)),
        out_shape=jax.ShapeDtypeStruct((N_TOK, D_INNER), jnp.float32),
    )(y0, y1)


def kernel(sequences, We, be, Wg, bg):
    N, P, D = sequences.shape
    flat = sequences.reshape(-1, D)
    top2, var, flat_p, pos, counts = _gating(flat, Wg, bg)

    counts = counts.reshape(N_EXPERTS)
    padded = ((counts + _BM - 1) // _BM) * _BM
    cum = jnp.cumsum(padded)
    pstart = cum - padded
    dest = pstart[top2] + pos
    dest_flat = dest.T.reshape(-1)
    starts = jnp.arange(_NB, dtype=jnp.int32) * _BM
    block_expert = jnp.minimum(
        jnp.sum((starts[:, None] >= cum[None, :]).astype(jnp.int32), axis=1),
        N_EXPERTS - 1).astype(jnp.int32)

    We_bf = _wcast(We)
    xs = _make_dispatch_scatter()(flat_p, dest_flat)
    ys = _gmm(block_expert, xs, We_bf, be)
    y0, y1 = _make_pair_gather()(ys, dest_flat)
    out = _combine(y0, y1)
    return (out.reshape(N, P, -1), var.reshape(N_EXPERTS))
